# Initial kernel scaffold; baseline (speedup 1.0000x reference)
import jax
import jax.numpy as jnp
from jax.experimental import pallas as pl


def kernel(x, edge_index, params):
    raise NotImplementedError("write your pallas kernel here")


# speedup vs baseline: 2.6326x; 2.6326x over previous
import functools

import jax
import jax.numpy as jnp
from jax import lax
from jax.experimental import pallas as pl
from jax.experimental.pallas import tpu as pltpu
from jax.experimental.pallas import tpu_sc as plsc

N_NODES = 10000
N_PAD = 10240
N_EDGES = 160000
E_PAD = 163840
N_WORKERS = 32
N_CHUNKS = 40
CHUNK = 128
ROWS_PER_TILE = 640
MAX_SEG_C = 128


@functools.lru_cache(maxsize=None)
def _make_seg_kernel(c):
    mesh = plsc.VectorSubcoreMesh(core_axis_name="c", subcore_axis_name="s")

    @functools.partial(
        pl.kernel,
        out_type=jax.ShapeDtypeStruct((2, N_PAD, c), jnp.float32),
        mesh=mesh,
        compiler_params=pltpu.CompilerParams(use_tc_tiling_on_sc=False),
        scratch_types=[
            pltpu.VMEM((N_CHUNKS, CHUNK), jnp.int32),
            pltpu.VMEM((N_CHUNKS, CHUNK), jnp.int32),
            pltpu.VMEM((CHUNK, c), jnp.float32),
            pltpu.VMEM_SHARED((N_PAD, c), jnp.float32),
            pltpu.SemaphoreType.DMA,
        ],
    )
    def seg(h_hbm, srcp_hbm, dstp_hbm, z_hbm, out_hbm, src_v, dst_v, buf_v, acc_sh, sem):
        cid = lax.axis_index("c")
        sid = lax.axis_index("s")
        wid = cid * 16 + sid

        pltpu.sync_copy(srcp_hbm.at[wid], src_v)
        pltpu.sync_copy(dstp_hbm.at[wid], dst_v)

        pltpu.sync_copy(z_hbm, acc_sh.at[pl.ds(sid * ROWS_PER_TILE, ROWS_PER_TILE)])
        plsc.subcore_barrier()

        def body(j, carry):
            pltpu.async_copy(h_hbm.at[src_v.at[j]], buf_v, sem).wait()
            pltpu.sync_copy(buf_v, acc_sh.at[dst_v.at[j]], add=True)
            return carry

        lax.fori_loop(0, N_CHUNKS, body, 0)
        plsc.subcore_barrier()

        pltpu.sync_copy(
            acc_sh.at[pl.ds(sid * ROWS_PER_TILE, ROWS_PER_TILE)],
            out_hbm.at[cid, pl.ds(sid * ROWS_PER_TILE, ROWS_PER_TILE)],
        )

    return seg


def _fused_tc(inputs, w=None, post_relu=False):
    n, c_in = inputs[0].shape
    c_out = w.shape[1] if w is not None else c_in
    bn = 1000
    n_in = len(inputs)

    def body(*refs):
        out_ref = refs[-1]
        acc = refs[0][...]
        for r in refs[1:n_in]:
            acc = acc + r[...]
        if w is not None:
            acc = jnp.dot(acc, refs[n_in][...], preferred_element_type=jnp.float32)
        if post_relu:
            acc = jnp.maximum(acc, 0.0)
        out_ref[...] = acc

    in_specs = [pl.BlockSpec((bn, c_in), lambda i: (i, 0)) for _ in inputs]
    args = list(inputs)
    if w is not None:
        in_specs.append(pl.BlockSpec(w.shape, lambda i: (0, 0)))
        args.append(w)
    return pl.pallas_call(
        body,
        grid=(n // bn,),
        in_specs=in_specs,
        out_specs=pl.BlockSpec((bn, c_out), lambda i: (i, 0)),
        out_shape=jax.ShapeDtypeStruct((n, c_out), jnp.float32),
    )(*args)


def _seg_chunked(h, srcp, dstp):
    c = h.shape[1]
    if c <= MAX_SEG_C:
        z = jnp.zeros((ROWS_PER_TILE, c), jnp.float32)
        out = _make_seg_kernel(c)(h, srcp, dstp, z)
        return out[0, :N_NODES], out[1, :N_NODES]
    p0s, p1s = [], []
    for lo in range(0, c, MAX_SEG_C):
        p0, p1 = _seg_chunked(h[:, lo:lo + MAX_SEG_C], srcp, dstp)
        p0s.append(p0)
        p1s.append(p1)
    return jnp.concatenate(p0s, axis=1), jnp.concatenate(p1s, axis=1)


def _reduction_matrix(c):
    return jnp.repeat(jnp.eye(c, dtype=jnp.float32), 2, axis=0)


def kernel(x, edge_index, params):
    src = edge_index[0]
    dst = edge_index[1]
    pad = E_PAD - N_EDGES
    srcp = jnp.concatenate([src, jnp.zeros((pad,), jnp.int32)]).reshape(
        N_WORKERS, N_CHUNKS, CHUNK)
    dstp = jnp.concatenate([dst, jnp.full((pad,), N_NODES, jnp.int32)]).reshape(
        N_WORKERS, N_CHUNKS, CHUNK)

    def mp(h, w, relu=True):
        c_in, c_out = w.shape
        if c_out < c_in:
            g = _fused_tc([h], w)
            p0, p1 = _seg_chunked(g, srcp, dstp)
            return _fused_tc([g, p0, p1], None, post_relu=relu)
        p0, p1 = _seg_chunked(h, srcp, dstp)
        return _fused_tc([h, p0, p1], w, post_relu=relu)

    def basic_block(h, w1, w2):
        t1 = mp(h, w1, relu=True)
        t2 = mp(t1, w2, relu=False)
        return _fused_tc([t2, h], None, post_relu=True)

    def ur_block(x_lateral, x_bottom, wt1, wt2, wm, winv):
        x_trans = basic_block(x_lateral, wt1, wt2)
        cat = jnp.concatenate([x_bottom, x_trans], axis=1)
        c = wm.shape[1]
        m = _fused_tc([cat], wm)
        p0, p1 = _seg_chunked(m, srcp, dstp)
        x_m = _fused_tc([m, p0, p1], None, post_relu=True)
        x_r = _fused_tc([cat], _reduction_matrix(c))
        ci, co = winv.shape
        if co < ci:
            g = _fused_tc([x_m, x_r], winv)
            q0, q1 = _seg_chunked(g, srcp, dstp)
            return _fused_tc([g, q0, q1], None, post_relu=True)
        y = _fused_tc([x_m, x_r], None)
        q0, q1 = _seg_chunked(y, srcp, dstp)
        return _fused_tc([y, q0, q1], winv, post_relu=True)

    p = params
    x1 = mp(mp(mp(x, p[0]), p[1]), p[2])
    x2 = mp(mp(mp(x1, p[3]), p[4]), p[5])
    x3 = mp(mp(mp(x2, p[6]), p[7]), p[8])
    x4 = mp(mp(mp(x3, p[9]), p[10]), p[11])
    u4 = ur_block(x4, x4, p[12], p[13], p[14], p[15])
    u3 = ur_block(x3, u4, p[16], p[17], p[18], p[19])
    u2 = ur_block(x2, u3, p[20], p[21], p[22], p[23])
    u1 = ur_block(x1, u2, p[24], p[25], p[26], p[27])
    return u1


# speedup vs baseline: 2.7912x; 1.0603x over previous
import functools

import jax
import jax.numpy as jnp
from jax import lax
from jax.experimental import pallas as pl
from jax.experimental.pallas import tpu as pltpu
from jax.experimental.pallas import tpu_sc as plsc

N_NODES = 10000
N_PAD = 10240
N_EDGES = 160000
E_PAD = 163840
N_WORKERS = 32
N_CHUNKS = 40
CHUNK = 128
ROWS_PER_TILE = 640
NBUF = 5
MAX_SEG_C = 64


@functools.lru_cache(maxsize=None)
def _make_seg_kernel(c):
    mesh = plsc.VectorSubcoreMesh(core_axis_name="c", subcore_axis_name="s")

    @functools.partial(
        pl.kernel,
        out_type=jax.ShapeDtypeStruct((2, N_PAD, c), jnp.float32),
        mesh=mesh,
        compiler_params=pltpu.CompilerParams(use_tc_tiling_on_sc=False),
        scratch_types=[
            pltpu.VMEM((N_CHUNKS, CHUNK), jnp.int32),
            pltpu.VMEM((N_CHUNKS, CHUNK), jnp.int32),
            pltpu.VMEM((NBUF, CHUNK, c), jnp.float32),
            pltpu.VMEM_SHARED((N_PAD, c), jnp.float32),
            pltpu.SemaphoreType.DMA((NBUF,)),
            pltpu.SemaphoreType.DMA((NBUF,)),
        ],
    )
    def seg(h_hbm, srcp_hbm, dstp_hbm, z_hbm, out_hbm, src_v, dst_v, buf_v, acc_sh,
            gsem, ssem):
        cid = lax.axis_index("c")
        sid = lax.axis_index("s")
        wid = cid * 16 + sid

        pltpu.sync_copy(srcp_hbm.at[wid], src_v)
        pltpu.sync_copy(dstp_hbm.at[wid], dst_v)

        pltpu.sync_copy(z_hbm, acc_sh.at[pl.ds(sid * ROWS_PER_TILE, ROWS_PER_TILE)])

        plsc.subcore_barrier()
        for b in range(NBUF):
            pltpu.async_copy(h_hbm.at[src_v.at[b]], buf_v.at[b], gsem.at[b])

        def wait_gather(b, j):
            pltpu.make_async_copy(z_hbm.at[pl.ds(0, CHUNK)], buf_v.at[b],
                                  gsem.at[b]).wait()

        def wait_scatter(b, j):
            pltpu.make_async_copy(z_hbm.at[pl.ds(0, CHUNK)], buf_v.at[b],
                                  ssem.at[b]).wait()

        def body(k, carry):
            for b in range(NBUF):
                j = k * NBUF + b
                wait_gather(b, j)
                pltpu.sync_copy(buf_v.at[b], acc_sh.at[dst_v.at[j]], add=True)
                pltpu.async_copy(h_hbm.at[src_v.at[j + NBUF]], buf_v.at[b],
                                 gsem.at[b])
            return carry

        lax.fori_loop(0, N_CHUNKS // NBUF - 1, body, 0)
        for b in range(NBUF):
            j = N_CHUNKS - NBUF + b
            wait_gather(b, j)
            pltpu.sync_copy(buf_v.at[b], acc_sh.at[dst_v.at[j]], add=True)
        plsc.subcore_barrier()

        pltpu.sync_copy(
            acc_sh.at[pl.ds(sid * ROWS_PER_TILE, ROWS_PER_TILE)],
            out_hbm.at[cid, pl.ds(sid * ROWS_PER_TILE, ROWS_PER_TILE)],
        )

    return seg


def _fused_tc(inputs, w=None, post_relu=False):
    n, c_in = inputs[0].shape
    c_out = w.shape[1] if w is not None else c_in
    bn = 1000
    n_in = len(inputs)

    def body(*refs):
        out_ref = refs[-1]
        acc = refs[0][...]
        for r in refs[1:n_in]:
            acc = acc + r[...]
        if w is not None:
            acc = jnp.dot(acc, refs[n_in][...], preferred_element_type=jnp.float32)
        if post_relu:
            acc = jnp.maximum(acc, 0.0)
        out_ref[...] = acc

    in_specs = [pl.BlockSpec((bn, c_in), lambda i: (i, 0)) for _ in inputs]
    args = list(inputs)
    if w is not None:
        in_specs.append(pl.BlockSpec(w.shape, lambda i: (0, 0)))
        args.append(w)
    return pl.pallas_call(
        body,
        grid=(n // bn,),
        in_specs=in_specs,
        out_specs=pl.BlockSpec((bn, c_out), lambda i: (i, 0)),
        out_shape=jax.ShapeDtypeStruct((n, c_out), jnp.float32),
    )(*args)


def _seg_chunked(h, srcp, dstp):
    c = h.shape[1]
    if c <= MAX_SEG_C:
        z = jnp.zeros((ROWS_PER_TILE, c), jnp.float32)
        out = _make_seg_kernel(c)(h, srcp, dstp, z)
        return out[0, :N_NODES], out[1, :N_NODES]
    p0s, p1s = [], []
    for lo in range(0, c, MAX_SEG_C):
        p0, p1 = _seg_chunked(h[:, lo:lo + MAX_SEG_C], srcp, dstp)
        p0s.append(p0)
        p1s.append(p1)
    return jnp.concatenate(p0s, axis=1), jnp.concatenate(p1s, axis=1)


def _reduction_matrix(c):
    return jnp.repeat(jnp.eye(c, dtype=jnp.float32), 2, axis=0)


def kernel(x, edge_index, params):
    src = edge_index[0]
    dst = edge_index[1]
    pad = E_PAD - N_EDGES
    srcp = jnp.concatenate([src, jnp.zeros((pad,), jnp.int32)]).reshape(
        N_WORKERS, N_CHUNKS, CHUNK)
    dstp = jnp.concatenate([dst, jnp.full((pad,), N_NODES, jnp.int32)]).reshape(
        N_WORKERS, N_CHUNKS, CHUNK)

    def mp(h, w, relu=True):
        c_in, c_out = w.shape
        if c_out < c_in:
            g = _fused_tc([h], w)
            p0, p1 = _seg_chunked(g, srcp, dstp)
            return _fused_tc([g, p0, p1], None, post_relu=relu)
        p0, p1 = _seg_chunked(h, srcp, dstp)
        return _fused_tc([h, p0, p1], w, post_relu=relu)

    def basic_block(h, w1, w2):
        t1 = mp(h, w1, relu=True)
        t2 = mp(t1, w2, relu=False)
        return _fused_tc([t2, h], None, post_relu=True)

    def ur_block(x_lateral, x_bottom, wt1, wt2, wm, winv):
        x_trans = basic_block(x_lateral, wt1, wt2)
        cat = jnp.concatenate([x_bottom, x_trans], axis=1)
        c = wm.shape[1]
        m = _fused_tc([cat], wm)
        p0, p1 = _seg_chunked(m, srcp, dstp)
        x_m = _fused_tc([m, p0, p1], None, post_relu=True)
        x_r = _fused_tc([cat], _reduction_matrix(c))
        ci, co = winv.shape
        if co < ci:
            g = _fused_tc([x_m, x_r], winv)
            q0, q1 = _seg_chunked(g, srcp, dstp)
            return _fused_tc([g, q0, q1], None, post_relu=True)
        y = _fused_tc([x_m, x_r], None)
        q0, q1 = _seg_chunked(y, srcp, dstp)
        return _fused_tc([y, q0, q1], winv, post_relu=True)

    p = params
    x1 = mp(mp(mp(x, p[0]), p[1]), p[2])
    x2 = mp(mp(mp(x1, p[3]), p[4]), p[5])
    x3 = mp(mp(mp(x2, p[6]), p[7]), p[8])
    x4 = mp(mp(mp(x3, p[9]), p[10]), p[11])
    u4 = ur_block(x4, x4, p[12], p[13], p[14], p[15])
    u3 = ur_block(x3, u4, p[16], p[17], p[18], p[19])
    u2 = ur_block(x2, u3, p[20], p[21], p[22], p[23])
    u1 = ur_block(x1, u2, p[24], p[25], p[26], p[27])
    return u1
